# baseline (speedup 1.0000x reference)
import functools
import math

import numpy as np

import jax
import jax.numpy as jnp
from jax import lax
from jax.experimental import pallas as pl
from jax.experimental.pallas import tpu as pltpu
from jax.experimental.pallas import tpu_sc as plsc

VOCAB = 100000
D_MODEL = 512
BATCH = 4
SEQ_LEN = 4096

NC = 2
NS = 16
NW = NC * NS
SEQ_PER_W = SEQ_LEN // NW
GROUP_ROWS = 16
NGROUP = SEQ_PER_W // GROUP_ROWS
NRING = 3
SCALE = math.sqrt(D_MODEL)
LANES = 16
VECS_PER_ROW = D_MODEL // LANES

_FREQ_EXP = -math.log(10000.0) / (D_MODEL // 2)
_TWO_PI = 2.0 * math.pi
_INV_TWO_PI = 1.0 / _TWO_PI
_MAGIC = 1.5 * 2.0 ** 23


def _sin_poly_coeffs():
    x = np.linspace(1e-6, np.pi, 4001)
    a = np.stack([x ** (2 * k + 1) for k in range(5)], axis=1)
    c, *_ = np.linalg.lstsq(a, np.sin(x), rcond=None)
    return [float(v) for v in c]


_C0, _C1, _C2, _C3, _C4 = _sin_poly_coeffs()


_mesh = plsc.VectorSubcoreMesh(core_axis_name="c", subcore_axis_name="s")


@functools.partial(
    pl.kernel,
    mesh=_mesh,
    out_type=jax.ShapeDtypeStruct((BATCH * SEQ_LEN, D_MODEL), jnp.float32),
    scratch_types=[
        pltpu.VMEM((BATCH, SEQ_PER_W), jnp.int32),
        pltpu.VMEM((D_MODEL,), jnp.float32),
        pltpu.VMEM((D_MODEL,), jnp.float32),
        pltpu.VMEM((NRING, BATCH, GROUP_ROWS, D_MODEL), jnp.float32),
        pltpu.SemaphoreType.DMA,
        pltpu.SemaphoreType.DMA,
        pltpu.SemaphoreType.DMA,
        pltpu.SemaphoreType.DMA,
        pltpu.SemaphoreType.DMA,
        pltpu.SemaphoreType.DMA,
        pltpu.SemaphoreType.DMA,
    ],
)
def _embed_sc(table_hbm, idx_hbm, out_hbm,
              idx_v, freq_v, phase_v, rows_v,
              isem, gsem0, gsem1, gsem2, wsem0, wsem1, wsem2):
    wid = lax.axis_index("s") * NC + lax.axis_index("c")
    seq_base = wid * SEQ_PER_W

    icp = pltpu.async_copy(idx_hbm.at[:, pl.ds(seq_base, SEQ_PER_W)],
                           idx_v, isem)

    lane = lax.iota(jnp.int32, 16)

    @plsc.parallel_loop(0, VECS_PER_ROW, 1, unroll=1)
    def mk_tables(j):
        d = j * LANES + lane
        half = lax.shift_right_logical(d, 1).astype(jnp.float32)
        sl = pl.ds(j * LANES, LANES)
        freq_v[sl] = jnp.exp(half * _FREQ_EXP)
        phase_v[sl] = (d & 1).astype(jnp.float32) * (math.pi / 2.0)

    icp.wait()

    gsems = (gsem0, gsem1, gsem2)
    wsems = (wsem0, wsem1, wsem2)

    def start_group(g):
        slot = g % NRING
        return tuple(
            pltpu.async_copy(
                table_hbm.at[idx_v.at[b, pl.ds(g * GROUP_ROWS, GROUP_ROWS)]],
                rows_v.at[slot, b], gsems[slot])
            for b in range(BATCH))

    pend = [None] * NGROUP
    wdesc = [None] * NGROUP

    for g in range(NRING):
        pend[g] = start_group(g)

    for g in range(NGROUP):
        slot = g % NRING
        for d in pend[g]:
            d.wait()

        row_refs = tuple(rows_v.at[slot, b] for b in range(BATCH))
        p0 = (seq_base + g * GROUP_ROWS).astype(jnp.float32)

        @plsc.parallel_loop(0, GROUP_ROWS * VECS_PER_ROW, 1, unroll=2)
        def body(i, row_refs=row_refs, p0=p0):
            r = lax.shift_right_logical(i, 5)
            sl = pl.ds((i & (VECS_PER_ROW - 1)) * LANES, LANES)
            p = p0 + r.astype(jnp.float32)
            ang = p * freq_v[sl] + phase_v[sl]
            k = (ang * _INV_TWO_PI + _MAGIC) - _MAGIC
            x = ang - k * _TWO_PI
            x2 = x * x
            h = _C4
            h = h * x2 + _C3
            h = h * x2 + _C2
            h = h * x2 + _C1
            h = h * x2 + _C0
            pe_vec = x * h
            for rr in row_refs:
                rr[r, sl] = rr[r, sl] * SCALE + pe_vec

        row0 = seq_base + g * GROUP_ROWS
        wdesc[g] = tuple(
            pltpu.async_copy(row_refs[b],
                             out_hbm.at[pl.ds(b * SEQ_LEN + row0, GROUP_ROWS)],
                             wsems[slot])
            for b in range(BATCH))

        pre = g + NRING - 1
        if NRING <= pre < NGROUP:
            old = pre - NRING
            for d in wdesc[old]:
                d.wait()
            pend[pre] = start_group(pre)

    for g in range(NGROUP - NRING, NGROUP):
        for d in wdesc[g]:
            d.wait()


def kernel(token_ids, W):
    out = _embed_sc(W, token_ids.astype(jnp.int32))
    return out.reshape(BATCH, SEQ_LEN, D_MODEL)
